# baseline (speedup 1.0000x reference)
import jax
import jax.numpy as jnp
from jax.experimental import pallas as pl

L = 2048
N_BASES = 4
BR = 256


def _body(seq_blk_ref, seq_row_ref, pairs_blk_ref, bt_ref, seq_out_ref, idx_out_ref):
    bt = bt_ref[:, :]
    sc = jnp.transpose(seq_blk_ref[:, :], (1, 0))
    sr = seq_row_ref[:, :]
    pc = jnp.transpose(pairs_blk_ref[:, :], (1, 0))

    for c in range(N_BASES):
        colv = jnp.zeros((BR, 1), jnp.float32)
        rowv = jnp.zeros((1, L), jnp.float32)
        for b in range(N_BASES):
            colv = colv + jnp.where(sc == b, bt[b, c], 0.0)
            rowv = rowv + jnp.where(sr == b, bt[b, c], 0.0)
        seq_out_ref[0, c, :, :] = jnp.broadcast_to(colv, (BR, L))
        seq_out_ref[0, N_BASES + c, :, :] = jnp.broadcast_to(rowv, (BR, L))

    jidx = jax.lax.broadcasted_iota(jnp.int32, (BR, L), 1)
    idx_out_ref[0, 0, :, :] = (jidx == pc).astype(jnp.float32)


@jax.jit
def kernel(seq, pairs, base_table):
    seq_row = seq.reshape(1, L)
    pairs_row = pairs.reshape(1, L)

    grid = (L // BR,)
    seq_ret, idx_ret = pl.pallas_call(
        _body,
        grid=grid,
        in_specs=[
            pl.BlockSpec((1, BR), lambda r: (0, r)),
            pl.BlockSpec((1, L), lambda r: (0, 0)),
            pl.BlockSpec((1, BR), lambda r: (0, r)),
            pl.BlockSpec((N_BASES, N_BASES), lambda r: (0, 0)),
        ],
        out_specs=[
            pl.BlockSpec((1, 2 * N_BASES, BR, L), lambda r: (0, 0, r, 0)),
            pl.BlockSpec((1, 1, BR, L), lambda r: (0, 0, r, 0)),
        ],
        out_shape=[
            jax.ShapeDtypeStruct((1, 2 * N_BASES, L, L), jnp.float32),
            jax.ShapeDtypeStruct((1, 1, L, L), jnp.float32),
        ],
    )(seq_row, seq_row, pairs_row, base_table)
    return seq_ret, idx_ret


# speedup vs baseline: 1.0158x; 1.0158x over previous
import jax
import jax.numpy as jnp
from jax.experimental import pallas as pl

L = 2048
N_BASES = 4
BR = 128


def _body(seq_blk_ref, seq_row_ref, pairs_blk_ref, bt_ref, seq_out_ref, idx_out_ref):
    bt = bt_ref[:, :]
    sc = jnp.transpose(seq_blk_ref[:, :], (1, 0))
    sr = seq_row_ref[:, :]
    pc = jnp.transpose(pairs_blk_ref[:, :], (1, 0))

    for c in range(N_BASES):
        colv = jnp.zeros((BR, 1), jnp.float32)
        rowv = jnp.zeros((1, L), jnp.float32)
        for b in range(N_BASES):
            colv = colv + jnp.where(sc == b, bt[b, c], 0.0)
            rowv = rowv + jnp.where(sr == b, bt[b, c], 0.0)
        seq_out_ref[0, c, :, :] = jnp.broadcast_to(colv, (BR, L))
        seq_out_ref[0, N_BASES + c, :, :] = jnp.broadcast_to(rowv, (BR, L))

    jidx = jax.lax.broadcasted_iota(jnp.int32, (BR, L), 1)
    idx_out_ref[0, 0, :, :] = (jidx == pc).astype(jnp.float32)


@jax.jit
def kernel(seq, pairs, base_table):
    seq_row = seq.reshape(1, L)
    pairs_row = pairs.reshape(1, L)

    grid = (L // BR,)
    seq_ret, idx_ret = pl.pallas_call(
        _body,
        grid=grid,
        in_specs=[
            pl.BlockSpec((1, BR), lambda r: (0, r)),
            pl.BlockSpec((1, L), lambda r: (0, 0)),
            pl.BlockSpec((1, BR), lambda r: (0, r)),
            pl.BlockSpec((N_BASES, N_BASES), lambda r: (0, 0)),
        ],
        out_specs=[
            pl.BlockSpec((1, 2 * N_BASES, BR, L), lambda r: (0, 0, r, 0)),
            pl.BlockSpec((1, 1, BR, L), lambda r: (0, 0, r, 0)),
        ],
        out_shape=[
            jax.ShapeDtypeStruct((1, 2 * N_BASES, L, L), jnp.float32),
            jax.ShapeDtypeStruct((1, 1, L, L), jnp.float32),
        ],
    )(seq_row, seq_row, pairs_row, base_table)
    return seq_ret, idx_ret
